# Initial kernel scaffold; baseline (speedup 1.0000x reference)
import functools

import jax
import jax.numpy as jnp
from jax import lax
from jax.experimental import pallas as pl
from jax.experimental.pallas import tpu as pltpu
from jax.experimental.pallas import tpu_sc as plsc

N_NODES = 10000
N_EDGES = 320000
D_FEAT = 128
D_EDGE = 16

G = 128
N_GROUPS = N_EDGES // G
NC = 2
NS = 16
NW = NC * NS


def _tc_tables(x_ref, w_ref, b_ref, p_ref, q_ref):
    res = jnp.dot(x_ref[...], w_ref[...], preferred_element_type=jnp.float32)
    p_ref[...] = res[:, :D_EDGE]
    q_ref[...] = res[:, D_EDGE:] + b_ref[...]


_mesh = plsc.VectorSubcoreMesh(core_axis_name="c", subcore_axis_name="s")


@functools.partial(
    pl.kernel,
    mesh=_mesh,
    out_type=jax.ShapeDtypeStruct((N_EDGES, D_EDGE), jnp.float32),
    scratch_types=[
        pltpu.VMEM((G,), jnp.int32),
        pltpu.VMEM((G,), jnp.int32),
        pltpu.VMEM((G, D_EDGE), jnp.float32),
        pltpu.VMEM((G, D_EDGE), jnp.float32),
        pltpu.SemaphoreType.DMA,
        pltpu.SemaphoreType.DMA,
    ],
)
def _sc_gather_add(p_hbm, q_hbm, src_hbm, dst_hbm, out_hbm,
                   sidx, didx, pbuf, qbuf, sem_p, sem_q):
    wid = lax.axis_index("s") * NC + lax.axis_index("c")
    n_w = (N_GROUPS // NW) + jnp.where(wid < (N_GROUPS % NW), 1, 0)

    def body(k, carry):
        g = wid + k * NW
        base = g * G
        pltpu.sync_copy(src_hbm.at[pl.ds(base, G)], sidx)
        pltpu.sync_copy(dst_hbm.at[pl.ds(base, G)], didx)
        cp = pltpu.async_copy(p_hbm.at[sidx], pbuf, sem_p)
        cq = pltpu.async_copy(q_hbm.at[didx], qbuf, sem_q)
        cp.wait()
        cq.wait()

        def add_row(i, c):
            pbuf[i, :] = pbuf[i, :] + qbuf[i, :]
            return c

        lax.fori_loop(0, G, add_row, 0, unroll=8)
        pltpu.sync_copy(pbuf, out_hbm.at[pl.ds(base, G)])
        return carry

    lax.fori_loop(0, n_w, body, 0)


def kernel(x, edge_index, pos, W, b):
    wcat = jnp.concatenate([W[:D_FEAT, :], W[D_FEAT:, :]], axis=1)
    p, q = pl.pallas_call(
        _tc_tables,
        out_shape=[
            jax.ShapeDtypeStruct((N_NODES, D_EDGE), jnp.float32),
            jax.ShapeDtypeStruct((N_NODES, D_EDGE), jnp.float32),
        ],
    )(x, wcat, b.reshape(1, D_EDGE))
    return _sc_gather_add(p, q, edge_index[0], edge_index[1])


# speedup vs baseline: 4.4808x; 4.4808x over previous
import functools

import jax
import jax.numpy as jnp
from jax import lax
from jax.experimental import pallas as pl
from jax.experimental.pallas import tpu as pltpu
from jax.experimental.pallas import tpu_sc as plsc

N_NODES = 10000
N_EDGES = 320000
D_FEAT = 128
D_EDGE = 16

G = 128
N_GROUPS = N_EDGES // G
NC = 2
NS = 16
NW = NC * NS


def _tc_tables(x_ref, w_ref, b_ref, p_ref, q_ref):
    res = jnp.dot(x_ref[...], w_ref[...], preferred_element_type=jnp.float32)
    p_ref[...] = res[:, :D_EDGE]
    q_ref[...] = res[:, D_EDGE:] + b_ref[...]


_mesh = plsc.VectorSubcoreMesh(core_axis_name="c", subcore_axis_name="s")


@functools.partial(
    pl.kernel,
    mesh=_mesh,
    compiler_params=pltpu.CompilerParams(use_tc_tiling_on_sc=False),
    out_type=jax.ShapeDtypeStruct((N_EDGES, D_EDGE), jnp.float32),
    scratch_types=[
        pltpu.VMEM((G,), jnp.int32),
        pltpu.VMEM((G,), jnp.int32),
        pltpu.VMEM((G, D_EDGE), jnp.float32),
        pltpu.VMEM((G, D_EDGE), jnp.float32),
        pltpu.SemaphoreType.DMA,
        pltpu.SemaphoreType.DMA,
    ],
)
def _sc_gather_add(p_hbm, q_hbm, src_hbm, dst_hbm, out_hbm,
                   sidx, didx, pbuf, qbuf, sem_p, sem_q):
    wid = lax.axis_index("s") * NC + lax.axis_index("c")
    n_w = (N_GROUPS // NW) + jnp.where(wid < (N_GROUPS % NW), 1, 0)

    def body(k, carry):
        g = wid + k * NW
        base = g * G
        pltpu.sync_copy(src_hbm.at[pl.ds(base, G)], sidx)
        pltpu.sync_copy(dst_hbm.at[pl.ds(base, G)], didx)
        cp = pltpu.async_copy(p_hbm.at[sidx], pbuf, sem_p)
        cq = pltpu.async_copy(q_hbm.at[didx], qbuf, sem_q)
        cp.wait()
        cq.wait()

        def add_row(i, c):
            pbuf[i, :] = pbuf[i, :] + qbuf[i, :]
            return c

        lax.fori_loop(0, G, add_row, 0, unroll=8)
        pltpu.sync_copy(pbuf, out_hbm.at[pl.ds(base, G)])
        return carry

    lax.fori_loop(0, n_w, body, 0)


def kernel(x, edge_index, pos, W, b):
    wcat = jnp.concatenate([W[:D_FEAT, :], W[D_FEAT:, :]], axis=1)
    p, q = pl.pallas_call(
        _tc_tables,
        out_shape=[
            jax.ShapeDtypeStruct((N_NODES, D_EDGE), jnp.float32),
            jax.ShapeDtypeStruct((N_NODES, D_EDGE), jnp.float32),
        ],
    )(x, wcat, b.reshape(1, D_EDGE))
    return _sc_gather_add(p, q, edge_index[0], edge_index[1])


# speedup vs baseline: 6.8137x; 1.5207x over previous
import functools

import jax
import jax.numpy as jnp
from jax import lax
from jax.experimental import pallas as pl
from jax.experimental.pallas import tpu as pltpu
from jax.experimental.pallas import tpu_sc as plsc

N_NODES = 10000
N_EDGES = 320000
D_FEAT = 128
D_EDGE = 16

NC = 2
NS = 16
NW = NC * NS
EPW = N_EDGES // NW
G = 128
NG = EPW // G
TAIL = EPW - NG * G


def _tc_tables(x_ref, w_ref, b_ref, p_ref, q_ref):
    res = jnp.dot(x_ref[...], w_ref[...], preferred_element_type=jnp.float32)
    p_ref[...] = res[:, :D_EDGE]
    q_ref[...] = res[:, D_EDGE:] + b_ref[...]


_mesh = plsc.VectorSubcoreMesh(core_axis_name="c", subcore_axis_name="s")


@functools.partial(
    pl.kernel,
    mesh=_mesh,
    compiler_params=pltpu.CompilerParams(use_tc_tiling_on_sc=False),
    out_type=jax.ShapeDtypeStruct((N_EDGES, D_EDGE), jnp.float32),
    scratch_types=[
        pltpu.VMEM((EPW,), jnp.int32),
        pltpu.VMEM((EPW,), jnp.int32),
        pltpu.VMEM((2, G, D_EDGE), jnp.float32),
        pltpu.VMEM((2, G, D_EDGE), jnp.float32),
        pltpu.VMEM((2, G, D_EDGE), jnp.float32),
        pltpu.SemaphoreType.DMA,
        pltpu.SemaphoreType.DMA,
        pltpu.SemaphoreType.DMA,
        pltpu.SemaphoreType.DMA,
        pltpu.SemaphoreType.DMA,
    ],
)
def _sc_gather_add(p_hbm, q_hbm, ei_hbm, out_hbm,
                   sidx, didx, pbuf, qbuf, obuf,
                   sem_i, sem_ga, sem_gb, sem_oa, sem_ob):
    wid = lax.axis_index("s") * NC + lax.axis_index("c")
    base_w = wid * EPW
    sem_g = (sem_ga, sem_gb)
    sem_o = (sem_oa, sem_ob)

    ci0 = pltpu.async_copy(ei_hbm.at[0, pl.ds(base_w, EPW)], sidx, sem_i)
    ci1 = pltpu.async_copy(ei_hbm.at[1, pl.ds(base_w, EPW)], didx, sem_i)
    ci0.wait()
    ci1.wait()

    def fire_gathers(g, s, n=G):
        pltpu.async_copy(p_hbm.at[sidx.at[pl.ds(g * G, n)]],
                         pbuf.at[s, pl.ds(0, n)], sem_g[s])
        pltpu.async_copy(q_hbm.at[didx.at[pl.ds(g * G, n)]],
                         qbuf.at[s, pl.ds(0, n)], sem_g[s])

    def wait_gathers(s, n=G):
        pltpu.make_async_copy(p_hbm.at[sidx.at[pl.ds(0, n)]],
                              pbuf.at[s, pl.ds(0, n)], sem_g[s]).wait()
        pltpu.make_async_copy(q_hbm.at[didx.at[pl.ds(0, n)]],
                              qbuf.at[s, pl.ds(0, n)], sem_g[s]).wait()

    def add_rows(s, n=G):
        def row(i, c):
            obuf[s, i, :] = pbuf[s, i, :] + qbuf[s, i, :]
            return c
        lax.fori_loop(0, n, row, 0, unroll=8)

    def fire_store(g, s, n=G):
        pltpu.async_copy(obuf.at[s, pl.ds(0, n)],
                         out_hbm.at[pl.ds(base_w + g * G, n)], sem_o[s])

    def wait_store(s, n=G):
        pltpu.make_async_copy(obuf.at[s, pl.ds(0, n)],
                              out_hbm.at[pl.ds(base_w, n)], sem_o[s]).wait()

    def pair(m, first, fire_next):
        g0 = 2 * m
        fire_gathers(g0 + 1, 1)
        wait_gathers(0)
        if not first:
            wait_store(0)
        add_rows(0)
        fire_store(g0, 0)
        if fire_next:
            fire_gathers(g0 + 2, 0)
        wait_gathers(1)
        if not first:
            wait_store(1)
        add_rows(1)
        fire_store(g0 + 1, 1)

    fire_gathers(0, 0)
    pair(0, first=True, fire_next=True)

    def body(m, c):
        pair(m, first=False, fire_next=True)
        return c

    lax.fori_loop(1, NG // 2 - 1, body, 0)
    pair(NG // 2 - 1, first=False, fire_next=False)

    fire_gathers(NG, 0, n=TAIL)
    wait_gathers(0, n=TAIL)
    wait_store(0)
    add_rows(0, n=TAIL)
    fire_store(NG, 0, n=TAIL)
    wait_store(0, n=TAIL)
    wait_store(1)


def kernel(x, edge_index, pos, W, b):
    wcat = jnp.concatenate([W[:D_FEAT, :], W[D_FEAT:, :]], axis=1)
    p, q = pl.pallas_call(
        _tc_tables,
        out_shape=[
            jax.ShapeDtypeStruct((N_NODES, D_EDGE), jnp.float32),
            jax.ShapeDtypeStruct((N_NODES, D_EDGE), jnp.float32),
        ],
    )(x, wcat, b.reshape(1, D_EDGE))
    return _sc_gather_add(p, q, edge_index)


# speedup vs baseline: 6.8321x; 1.0027x over previous
import functools

import jax
import jax.numpy as jnp
from jax import lax
from jax.experimental import pallas as pl
from jax.experimental.pallas import tpu as pltpu
from jax.experimental.pallas import tpu_sc as plsc

N_NODES = 10000
N_EDGES = 320000
D_FEAT = 128
D_EDGE = 16

NC = 2
NS = 16
NW = NC * NS
EPW = N_EDGES // NW
G = 128
NG = EPW // G
TAIL = EPW - NG * G


def _tc_tables(x_ref, w_ref, b_ref, p_ref, q_ref):
    res = jnp.dot(x_ref[...], w_ref[...], preferred_element_type=jnp.float32)
    p_ref[...] = res[:, :D_EDGE]
    q_ref[...] = res[:, D_EDGE:] + b_ref[...]


_mesh = plsc.VectorSubcoreMesh(core_axis_name="c", subcore_axis_name="s")


@functools.partial(
    pl.kernel,
    mesh=_mesh,
    compiler_params=pltpu.CompilerParams(use_tc_tiling_on_sc=False),
    out_type=jax.ShapeDtypeStruct((N_EDGES * D_EDGE,), jnp.float32),
    scratch_types=[
        pltpu.VMEM((EPW,), jnp.int32),
        pltpu.VMEM((EPW,), jnp.int32),
        pltpu.VMEM((2, G, D_EDGE), jnp.float32),
        pltpu.VMEM((2, G, D_EDGE), jnp.float32),
        pltpu.VMEM((2, G * D_EDGE), jnp.float32),
        pltpu.SemaphoreType.DMA,
        pltpu.SemaphoreType.DMA,
        pltpu.SemaphoreType.DMA,
        pltpu.SemaphoreType.DMA,
        pltpu.SemaphoreType.DMA,
    ],
)
def _sc_gather_add(p_hbm, q_hbm, ei_hbm, out_hbm,
                   sidx, didx, pbuf, qbuf, obuf,
                   sem_i, sem_ga, sem_gb, sem_oa, sem_ob):
    wid = lax.axis_index("s") * NC + lax.axis_index("c")
    base_w = wid * EPW
    sem_g = (sem_ga, sem_gb)
    sem_o = (sem_oa, sem_ob)

    ci0 = pltpu.async_copy(ei_hbm.at[0, pl.ds(base_w, EPW)], sidx, sem_i)
    ci1 = pltpu.async_copy(ei_hbm.at[1, pl.ds(base_w, EPW)], didx, sem_i)
    ci0.wait()
    ci1.wait()

    def fire_gathers(g, s, n=G):
        pltpu.async_copy(p_hbm.at[sidx.at[pl.ds(g * G, n)]],
                         pbuf.at[s, pl.ds(0, n)], sem_g[s])
        pltpu.async_copy(q_hbm.at[didx.at[pl.ds(g * G, n)]],
                         qbuf.at[s, pl.ds(0, n)], sem_g[s])

    def wait_gathers(s, n=G):
        pltpu.make_async_copy(p_hbm.at[sidx.at[pl.ds(0, n)]],
                              pbuf.at[s, pl.ds(0, n)], sem_g[s]).wait()
        pltpu.make_async_copy(q_hbm.at[didx.at[pl.ds(0, n)]],
                              qbuf.at[s, pl.ds(0, n)], sem_g[s]).wait()

    def add_rows(s, n=G):
        def row(i, c):
            obuf[s, pl.ds(i * D_EDGE, D_EDGE)] = pbuf[s, i, :] + qbuf[s, i, :]
            return c
        lax.fori_loop(0, n, row, 0, unroll=8)

    def fire_store(g, s, n=G):
        pltpu.async_copy(obuf.at[s, pl.ds(0, n * D_EDGE)],
                         out_hbm.at[pl.ds((base_w + g * G) * D_EDGE, n * D_EDGE)],
                         sem_o[s])

    def wait_store(s, n=G):
        pltpu.make_async_copy(obuf.at[s, pl.ds(0, n * D_EDGE)],
                              out_hbm.at[pl.ds(base_w * D_EDGE, n * D_EDGE)],
                              sem_o[s]).wait()

    def pair(m, first, fire_next):
        g0 = 2 * m
        fire_gathers(g0 + 1, 1)
        wait_gathers(0)
        if not first:
            wait_store(0)
        add_rows(0)
        fire_store(g0, 0)
        if fire_next:
            fire_gathers(g0 + 2, 0)
        wait_gathers(1)
        if not first:
            wait_store(1)
        add_rows(1)
        fire_store(g0 + 1, 1)

    fire_gathers(0, 0)
    pair(0, first=True, fire_next=True)

    def body(m, c):
        pair(m, first=False, fire_next=True)
        return c

    lax.fori_loop(1, NG // 2 - 1, body, 0)
    pair(NG // 2 - 1, first=False, fire_next=False)

    fire_gathers(NG, 0, n=TAIL)
    wait_gathers(0, n=TAIL)
    wait_store(0)
    add_rows(0, n=TAIL)
    fire_store(NG, 0, n=TAIL)
    wait_store(0, n=TAIL)
    wait_store(1)


def kernel(x, edge_index, pos, W, b):
    wcat = jnp.concatenate([W[:D_FEAT, :], W[D_FEAT:, :]], axis=1)
    p, q = pl.pallas_call(
        _tc_tables,
        out_shape=[
            jax.ShapeDtypeStruct((N_NODES, D_EDGE), jnp.float32),
            jax.ShapeDtypeStruct((N_NODES, D_EDGE), jnp.float32),
        ],
    )(x, wcat, b.reshape(1, D_EDGE))
    flat = _sc_gather_add(p, q, edge_index)
    return flat.reshape(N_EDGES, D_EDGE)
